# Initial kernel scaffold; baseline (speedup 1.0000x reference)
import jax
import jax.numpy as jnp
from jax.experimental import pallas as pl


def kernel(x, edge_dst, edge_src, W):
    raise NotImplementedError("write your pallas kernel here")


# speedup vs baseline: 6.0502x; 6.0502x over previous
import functools

import jax
import jax.numpy as jnp
from jax import lax
from jax.experimental import pallas as pl
from jax.experimental.pallas import tpu as pltpu
from jax.experimental.pallas import tpu_sc as plsc

N_NODES_ = 10000
N_EDGES_ = 320000
FEAT_ = 128
HFEAT_ = FEAT_ // 2
NC_ = 2
NS_ = 16
EDGES_PER_SUB_ = N_EDGES_ // NS_
CHUNK_ = 80
CHUNKS_ = EDGES_PER_SUB_ // CHUNK_
ROWS_PER_TILE_ = N_NODES_ // NS_
ZROWS_ = 25


def _sc_agg_body(x_hbm, src_hbm, dst_hbm, zero_hbm, zcnt_hbm, ones_hbm,
                 psum_hbm, pcnt_hbm,
                 src_v, dst_v, rows_v, ones_v, acc_sh, cnt_sh, sem):
    c = lax.axis_index("c")
    s = lax.axis_index("s")
    wid = c * NS_ + s

    @pl.loop(0, ROWS_PER_TILE_ // ZROWS_)
    def _(b):
        base = s * ROWS_PER_TILE_ + b * ZROWS_
        pltpu.sync_copy(zero_hbm, acc_sh.at[pl.ds(base, ZROWS_)])
        pltpu.sync_copy(zcnt_hbm, cnt_sh.at[pl.ds(base, ZROWS_)])

    pltpu.sync_copy(ones_hbm, ones_v)
    pltpu.sync_copy(src_hbm.at[s], src_v)
    pltpu.sync_copy(dst_hbm.at[s], dst_v)
    plsc.subcore_barrier()

    @pl.loop(0, CHUNKS_)
    def _(j):
        pltpu.async_copy(x_hbm.at[c].at[src_v.at[j]], rows_v, sem).wait()
        pltpu.sync_copy(rows_v, acc_sh.at[dst_v.at[j]], add=True)

        @pl.when(c == 0)
        def _():
            pltpu.sync_copy(ones_v, cnt_sh.at[dst_v.at[j]], add=True)

    plsc.subcore_barrier()

    base = s * ROWS_PER_TILE_
    pltpu.sync_copy(acc_sh.at[pl.ds(base, ROWS_PER_TILE_)], psum_hbm.at[wid])

    @pl.when(c == 0)
    def _():
        pltpu.sync_copy(cnt_sh.at[pl.ds(base, ROWS_PER_TILE_)],
                        pcnt_hbm.at[s])


_sc_agg = functools.partial(
    pl.kernel,
    out_type=(
        jax.ShapeDtypeStruct((NC_ * NS_, ROWS_PER_TILE_, HFEAT_), jnp.float32),
        jax.ShapeDtypeStruct((NS_, ROWS_PER_TILE_, 16), jnp.float32),
    ),
    mesh=plsc.VectorSubcoreMesh(core_axis_name="c", subcore_axis_name="s"),
    scratch_types=[
        pltpu.VMEM((CHUNKS_, CHUNK_), jnp.int32),
        pltpu.VMEM((CHUNKS_, CHUNK_), jnp.int32),
        pltpu.VMEM((CHUNK_, HFEAT_), jnp.float32),
        pltpu.VMEM((CHUNK_, 16), jnp.float32),
        pltpu.VMEM_SHARED((N_NODES_, HFEAT_), jnp.float32),
        pltpu.VMEM_SHARED((N_NODES_, 16), jnp.float32),
        pltpu.SemaphoreType.DMA,
    ],
    compiler_params=pltpu.CompilerParams(use_tc_tiling_on_sc=False),
)(_sc_agg_body)


def _tc_combine_body(x_ref, w_ref, ps_ref, pc_ref, o_ref):
    xb = x_ref[...]
    w = w_ref[...]
    ssum = jnp.concatenate([ps_ref[0], ps_ref[1]], axis=-1)
    cnt = pc_ref[:, 0:1]
    agg = ssum / jnp.maximum(cnt, 1.0)
    nr = jnp.dot(xb, w, preferred_element_type=jnp.float32,
                 precision=lax.Precision.HIGHEST)
    am = jnp.dot(agg, w, preferred_element_type=jnp.float32,
                 precision=lax.Precision.HIGHEST)
    o_ref[...] = jnp.maximum(jnp.concatenate([nr, am], axis=-1), 0.0)


_TC_ROWS = 2000


def _tc_combine(x2d, W, psum, pcnt):
    grid = (N_NODES_ // _TC_ROWS,)
    return pl.pallas_call(
        _tc_combine_body,
        grid=grid,
        in_specs=[
            pl.BlockSpec((_TC_ROWS, FEAT_), lambda i: (i, 0)),
            pl.BlockSpec((FEAT_, FEAT_), lambda i: (0, 0)),
            pl.BlockSpec((NC_, _TC_ROWS, HFEAT_), lambda i: (0, i, 0)),
            pl.BlockSpec((_TC_ROWS, 16), lambda i: (i, 0)),
        ],
        out_specs=pl.BlockSpec((_TC_ROWS, 2 * FEAT_), lambda i: (i, 0)),
        out_shape=jax.ShapeDtypeStruct((N_NODES_, 2 * FEAT_), jnp.float32),
    )(x2d, W, psum, pcnt)


def kernel(x, edge_dst, edge_src, W):
    x2d = x.astype(jnp.float32).reshape(N_NODES_, FEAT_)
    xh = jnp.stack([x2d[:, :HFEAT_], x2d[:, HFEAT_:]])
    src = edge_src.astype(jnp.int32).reshape(NS_, CHUNKS_, CHUNK_)
    dst = edge_dst.astype(jnp.int32).reshape(NS_, CHUNKS_, CHUNK_)
    zero = jnp.zeros((ZROWS_, HFEAT_), jnp.float32)
    zcnt = jnp.zeros((ZROWS_, 16), jnp.float32)
    ones = jnp.ones((CHUNK_, 16), jnp.float32)
    psum, pcnt = _sc_agg(xh, src, dst, zero, zcnt, ones)
    psum = psum.reshape(NC_, N_NODES_, HFEAT_)
    pcnt = pcnt.reshape(N_NODES_, 16)
    out = _tc_combine(x2d, W, psum, pcnt)
    return out.reshape(N_NODES_, 1, 1, 2 * FEAT_)


# speedup vs baseline: 9.2683x; 1.5319x over previous
import functools

import jax
import jax.numpy as jnp
from jax import lax
from jax.experimental import pallas as pl
from jax.experimental.pallas import tpu as pltpu
from jax.experimental.pallas import tpu_sc as plsc

N_NODES_ = 10000
N_EDGES_ = 320000
FEAT_ = 128
HFEAT_ = FEAT_ // 2
NC_ = 2
NS_ = 16
EDGES_PER_SUB_ = N_EDGES_ // NS_
CHUNK_ = 80
CHUNKS_ = EDGES_PER_SUB_ // CHUNK_
ROWS_PER_TILE_ = N_NODES_ // NS_
ZROWS_ = 25


def _sc_agg_body(x_hbm, src_hbm, dst_hbm, zero_hbm, zcnt_hbm, ones_hbm,
                 psum_hbm, pcnt_hbm,
                 src_v, dst_v, rows0_v, rows1_v, ones_v, acc_sh, cnt_sh,
                 sem0, sem1):
    c = lax.axis_index("c")
    s = lax.axis_index("s")
    wid = c * NS_ + s
    xv = x_hbm.at[c]
    rows = (rows0_v, rows1_v)
    sems = (sem0, sem1)

    @pl.loop(0, ROWS_PER_TILE_ // ZROWS_)
    def _(b):
        base = s * ROWS_PER_TILE_ + b * ZROWS_
        pltpu.sync_copy(zero_hbm, acc_sh.at[pl.ds(base, ZROWS_)])
        pltpu.sync_copy(zcnt_hbm, cnt_sh.at[pl.ds(base, ZROWS_)])

    pltpu.sync_copy(ones_hbm, ones_v)
    pltpu.sync_copy(src_hbm.at[s], src_v)
    pltpu.sync_copy(dst_hbm.at[s], dst_v)
    plsc.subcore_barrier()

    def process(m, b):
        pltpu.make_async_copy(xv.at[src_v.at[m]], rows[b], sems[b]).wait()
        pltpu.sync_copy(rows[b], acc_sh.at[dst_v.at[m]], add=True)
        @pl.when(c == b)
        def _():
            pltpu.sync_copy(ones_v, cnt_sh.at[dst_v.at[m]], add=True)

    for b in range(2):
        pltpu.async_copy(xv.at[src_v.at[b]], rows[b], sems[b])

    @pl.loop(2, CHUNKS_, step=2)
    def _(j):
        for b in range(2):
            process(j - 2 + b, b)
            pltpu.async_copy(xv.at[src_v.at[j + b]], rows[b], sems[b])

    for b in range(2):
        process(CHUNKS_ - 2 + b, b)

    plsc.subcore_barrier()

    base = s * ROWS_PER_TILE_
    pltpu.sync_copy(acc_sh.at[pl.ds(base, ROWS_PER_TILE_)], psum_hbm.at[wid])
    pltpu.sync_copy(cnt_sh.at[pl.ds(base, ROWS_PER_TILE_)], pcnt_hbm.at[wid])


_sc_agg = functools.partial(
    pl.kernel,
    out_type=(
        jax.ShapeDtypeStruct((NC_ * NS_, ROWS_PER_TILE_, HFEAT_), jnp.float32),
        jax.ShapeDtypeStruct((NC_ * NS_, ROWS_PER_TILE_, 16), jnp.float32),
    ),
    mesh=plsc.VectorSubcoreMesh(core_axis_name="c", subcore_axis_name="s"),
    scratch_types=[
        pltpu.VMEM((CHUNKS_, CHUNK_), jnp.int32),
        pltpu.VMEM((CHUNKS_, CHUNK_), jnp.int32),
        pltpu.VMEM((CHUNK_, HFEAT_), jnp.float32),
        pltpu.VMEM((CHUNK_, HFEAT_), jnp.float32),
        pltpu.VMEM((CHUNK_, 16), jnp.float32),
        pltpu.VMEM_SHARED((N_NODES_, HFEAT_), jnp.float32),
        pltpu.VMEM_SHARED((N_NODES_, 16), jnp.float32),
        pltpu.SemaphoreType.DMA,
        pltpu.SemaphoreType.DMA,
    ],
    compiler_params=pltpu.CompilerParams(use_tc_tiling_on_sc=False),
)(_sc_agg_body)


def _tc_combine_body(x_ref, w_ref, ps_ref, pc_ref, o_ref):
    xb = x_ref[...]
    w = w_ref[...]
    ssum = jnp.concatenate([ps_ref[0], ps_ref[1]], axis=-1)
    cnt = pc_ref[0, :, 0:1] + pc_ref[1, :, 0:1]
    agg = ssum / jnp.maximum(cnt, 1.0)
    nr = jnp.dot(xb, w, preferred_element_type=jnp.float32,
                 precision=lax.Precision.HIGHEST)
    am = jnp.dot(agg, w, preferred_element_type=jnp.float32,
                 precision=lax.Precision.HIGHEST)
    o_ref[...] = jnp.maximum(jnp.concatenate([nr, am], axis=-1), 0.0)


_TC_ROWS = 2000


def _tc_combine(x2d, W, psum, pcnt):
    grid = (N_NODES_ // _TC_ROWS,)
    return pl.pallas_call(
        _tc_combine_body,
        grid=grid,
        in_specs=[
            pl.BlockSpec((_TC_ROWS, FEAT_), lambda i: (i, 0)),
            pl.BlockSpec((FEAT_, FEAT_), lambda i: (0, 0)),
            pl.BlockSpec((NC_, _TC_ROWS, HFEAT_), lambda i: (0, i, 0)),
            pl.BlockSpec((NC_, _TC_ROWS, 16), lambda i: (0, i, 0)),
        ],
        out_specs=pl.BlockSpec((_TC_ROWS, 2 * FEAT_), lambda i: (i, 0)),
        out_shape=jax.ShapeDtypeStruct((N_NODES_, 2 * FEAT_), jnp.float32),
    )(x2d, W, psum, pcnt)


def kernel(x, edge_dst, edge_src, W):
    x2d = x.astype(jnp.float32).reshape(N_NODES_, FEAT_)
    xh = jnp.stack([x2d[:, :HFEAT_], x2d[:, HFEAT_:]])
    src = edge_src.astype(jnp.int32).reshape(NS_, CHUNKS_, CHUNK_)
    dst = edge_dst.astype(jnp.int32).reshape(NS_, CHUNKS_, CHUNK_)
    zero = jnp.zeros((ZROWS_, HFEAT_), jnp.float32)
    zcnt = jnp.zeros((ZROWS_, 16), jnp.float32)
    ones = jnp.ones((CHUNK_, 16), jnp.float32)
    psum, pcnt = _sc_agg(xh, src, dst, zero, zcnt, ones)
    psum = psum.reshape(NC_, N_NODES_, HFEAT_)
    pcnt = pcnt.reshape(NC_, N_NODES_, 16)
    out = _tc_combine(x2d, W, psum, pcnt)
    return out.reshape(N_NODES_, 1, 1, 2 * FEAT_)


# speedup vs baseline: 9.8134x; 1.0588x over previous
import functools

import jax
import jax.numpy as jnp
from jax import lax
from jax.experimental import pallas as pl
from jax.experimental.pallas import tpu as pltpu
from jax.experimental.pallas import tpu_sc as plsc

N_NODES_ = 10000
N_EDGES_ = 320000
FEAT_ = 128
HFEAT_ = FEAT_ // 2
NC_ = 2
NS_ = 16
EDGES_PER_SUB_ = N_EDGES_ // NS_
CHUNK_ = 125
CHUNKS_ = EDGES_PER_SUB_ // CHUNK_
ROWS_PER_TILE_ = N_NODES_ // NS_
ZROWS_ = 25


def _sc_agg_body(x_hbm, src_hbm, dst_hbm, zero_hbm, zcnt_hbm, ones_hbm,
                 psum_hbm, pcnt_hbm,
                 src_v, dst_v, rows0_v, rows1_v, ones_v, acc_sh, cnt_sh,
                 sem0, sem1):
    c = lax.axis_index("c")
    s = lax.axis_index("s")
    wid = c * NS_ + s
    xv = x_hbm.at[c]
    rows = (rows0_v, rows1_v)
    sems = (sem0, sem1)

    @pl.loop(0, ROWS_PER_TILE_ // ZROWS_)
    def _(b):
        base = s * ROWS_PER_TILE_ + b * ZROWS_
        pltpu.sync_copy(zero_hbm, acc_sh.at[pl.ds(base, ZROWS_)])
        pltpu.sync_copy(zcnt_hbm, cnt_sh.at[pl.ds(base, ZROWS_)])

    pltpu.sync_copy(ones_hbm, ones_v)
    pltpu.sync_copy(src_hbm.at[s], src_v)
    pltpu.sync_copy(dst_hbm.at[s], dst_v)
    plsc.subcore_barrier()

    def process(m, b):
        pltpu.make_async_copy(xv.at[src_v.at[m]], rows[b], sems[b]).wait()
        pltpu.sync_copy(rows[b], acc_sh.at[dst_v.at[m]], add=True)
        @pl.when(c == b)
        def _():
            pltpu.sync_copy(ones_v, cnt_sh.at[dst_v.at[m]], add=True)

    for b in range(2):
        pltpu.async_copy(xv.at[src_v.at[b]], rows[b], sems[b])

    @pl.loop(2, CHUNKS_, step=2)
    def _(j):
        for b in range(2):
            process(j - 2 + b, b)
            pltpu.async_copy(xv.at[src_v.at[j + b]], rows[b], sems[b])

    for b in range(2):
        process(CHUNKS_ - 2 + b, b)

    plsc.subcore_barrier()

    base = s * ROWS_PER_TILE_
    pltpu.sync_copy(acc_sh.at[pl.ds(base, ROWS_PER_TILE_)], psum_hbm.at[wid])
    pltpu.sync_copy(cnt_sh.at[pl.ds(base, ROWS_PER_TILE_)], pcnt_hbm.at[wid])


_sc_agg = functools.partial(
    pl.kernel,
    out_type=(
        jax.ShapeDtypeStruct((NC_ * NS_, ROWS_PER_TILE_, HFEAT_), jnp.float32),
        jax.ShapeDtypeStruct((NC_ * NS_, ROWS_PER_TILE_, 16), jnp.float32),
    ),
    mesh=plsc.VectorSubcoreMesh(core_axis_name="c", subcore_axis_name="s"),
    scratch_types=[
        pltpu.VMEM((CHUNKS_, CHUNK_), jnp.int32),
        pltpu.VMEM((CHUNKS_, CHUNK_), jnp.int32),
        pltpu.VMEM((CHUNK_, HFEAT_), jnp.float32),
        pltpu.VMEM((CHUNK_, HFEAT_), jnp.float32),
        pltpu.VMEM((CHUNK_, 16), jnp.float32),
        pltpu.VMEM_SHARED((N_NODES_, HFEAT_), jnp.float32),
        pltpu.VMEM_SHARED((N_NODES_, 16), jnp.float32),
        pltpu.SemaphoreType.DMA,
        pltpu.SemaphoreType.DMA,
    ],
    compiler_params=pltpu.CompilerParams(use_tc_tiling_on_sc=False),
)(_sc_agg_body)


def _tc_combine_body(x_ref, w_ref, ps_ref, pc_ref, o_ref):
    xb = x_ref[...]
    w = w_ref[...]
    ssum = jnp.concatenate([ps_ref[0], ps_ref[1]], axis=-1)
    cnt = pc_ref[0, :, 0:1] + pc_ref[1, :, 0:1]
    agg = ssum / jnp.maximum(cnt, 1.0)
    nr = jnp.dot(xb, w, preferred_element_type=jnp.float32,
                 precision=lax.Precision.HIGHEST)
    am = jnp.dot(agg, w, preferred_element_type=jnp.float32,
                 precision=lax.Precision.HIGHEST)
    o_ref[...] = jnp.maximum(jnp.concatenate([nr, am], axis=-1), 0.0)


_TC_ROWS = 2000


def _tc_combine(x2d, W, psum, pcnt):
    grid = (N_NODES_ // _TC_ROWS,)
    return pl.pallas_call(
        _tc_combine_body,
        grid=grid,
        in_specs=[
            pl.BlockSpec((_TC_ROWS, FEAT_), lambda i: (i, 0)),
            pl.BlockSpec((FEAT_, FEAT_), lambda i: (0, 0)),
            pl.BlockSpec((NC_, _TC_ROWS, HFEAT_), lambda i: (0, i, 0)),
            pl.BlockSpec((NC_, _TC_ROWS, 16), lambda i: (0, i, 0)),
        ],
        out_specs=pl.BlockSpec((_TC_ROWS, 2 * FEAT_), lambda i: (i, 0)),
        out_shape=jax.ShapeDtypeStruct((N_NODES_, 2 * FEAT_), jnp.float32),
    )(x2d, W, psum, pcnt)


def kernel(x, edge_dst, edge_src, W):
    x2d = x.astype(jnp.float32).reshape(N_NODES_, FEAT_)
    xh = jnp.stack([x2d[:, :HFEAT_], x2d[:, HFEAT_:]])
    src = edge_src.astype(jnp.int32).reshape(NS_, CHUNKS_, CHUNK_)
    dst = edge_dst.astype(jnp.int32).reshape(NS_, CHUNKS_, CHUNK_)
    zero = jnp.zeros((ZROWS_, HFEAT_), jnp.float32)
    zcnt = jnp.zeros((ZROWS_, 16), jnp.float32)
    ones = jnp.ones((CHUNK_, 16), jnp.float32)
    psum, pcnt = _sc_agg(xh, src, dst, zero, zcnt, ones)
    psum = psum.reshape(NC_, N_NODES_, HFEAT_)
    pcnt = pcnt.reshape(NC_, N_NODES_, 16)
    out = _tc_combine(x2d, W, psum, pcnt)
    return out.reshape(N_NODES_, 1, 1, 2 * FEAT_)


# speedup vs baseline: 13.8265x; 1.4089x over previous
import functools

import jax
import jax.numpy as jnp
from jax import lax
from jax.experimental import pallas as pl
from jax.experimental.pallas import tpu as pltpu
from jax.experimental.pallas import tpu_sc as plsc

N_NODES_ = 10000
N_EDGES_ = 320000
FEAT_ = 128
HFEAT_ = FEAT_ // 2
NC_ = 2
NS_ = 16
EDGES_PER_SUB_ = N_EDGES_ // NS_
CHUNK_ = 125
CHUNKS_ = EDGES_PER_SUB_ // CHUNK_
NBUF_ = 4
ROWS_PER_TILE_ = N_NODES_ // NS_


def _sc_agg_body(x_hbm, src_hbm, dst_hbm, zero_hbm, zcnt_hbm, ones_hbm,
                 psum_hbm, pcnt_hbm,
                 src_v, dst_v, rows0_v, rows1_v, rows2_v, rows3_v, ones_v,
                 acc_sh, cnt_sh, sem0, sem1, sem2, sem3):
    c = lax.axis_index("c")
    s = lax.axis_index("s")
    wid = c * NS_ + s
    xv = x_hbm.at[c]
    rows = (rows0_v, rows1_v, rows2_v, rows3_v)
    sems = (sem0, sem1, sem2, sem3)

    base = s * ROWS_PER_TILE_
    pltpu.sync_copy(zero_hbm, acc_sh.at[pl.ds(base, ROWS_PER_TILE_)])
    pltpu.sync_copy(zcnt_hbm, cnt_sh.at[pl.ds(base, ROWS_PER_TILE_)])

    pltpu.sync_copy(ones_hbm, ones_v)
    pltpu.sync_copy(src_hbm.at[s], src_v)
    pltpu.sync_copy(dst_hbm.at[s], dst_v)
    plsc.subcore_barrier()

    def process(m, b):
        pltpu.make_async_copy(xv.at[src_v.at[m]], rows[b], sems[b]).wait()
        pltpu.sync_copy(rows[b], acc_sh.at[dst_v.at[m]], add=True)
        @pl.when(c == b % 2)
        def _():
            pltpu.sync_copy(ones_v, cnt_sh.at[dst_v.at[m]], add=True)

    for b in range(NBUF_):
        pltpu.async_copy(xv.at[src_v.at[b]], rows[b], sems[b])

    @pl.loop(NBUF_, CHUNKS_, step=NBUF_)
    def _(j):
        for b in range(NBUF_):
            process(j - NBUF_ + b, b)
            pltpu.async_copy(xv.at[src_v.at[j + b]], rows[b], sems[b])

    for b in range(NBUF_):
        process(CHUNKS_ - NBUF_ + b, b)

    plsc.subcore_barrier()

    pltpu.sync_copy(acc_sh.at[pl.ds(base, ROWS_PER_TILE_)], psum_hbm.at[wid])
    pltpu.sync_copy(cnt_sh.at[pl.ds(base, ROWS_PER_TILE_)], pcnt_hbm.at[wid])


_sc_agg = functools.partial(
    pl.kernel,
    out_type=(
        jax.ShapeDtypeStruct((NC_ * NS_, ROWS_PER_TILE_, HFEAT_), jnp.float32),
        jax.ShapeDtypeStruct((NC_ * NS_, ROWS_PER_TILE_, 16), jnp.float32),
    ),
    mesh=plsc.VectorSubcoreMesh(core_axis_name="c", subcore_axis_name="s"),
    scratch_types=[
        pltpu.VMEM((CHUNKS_, CHUNK_), jnp.int32),
        pltpu.VMEM((CHUNKS_, CHUNK_), jnp.int32),
        pltpu.VMEM((CHUNK_, HFEAT_), jnp.float32),
        pltpu.VMEM((CHUNK_, HFEAT_), jnp.float32),
        pltpu.VMEM((CHUNK_, HFEAT_), jnp.float32),
        pltpu.VMEM((CHUNK_, HFEAT_), jnp.float32),
        pltpu.VMEM((CHUNK_, 16), jnp.float32),
        pltpu.VMEM_SHARED((N_NODES_, HFEAT_), jnp.float32),
        pltpu.VMEM_SHARED((N_NODES_, 16), jnp.float32),
        pltpu.SemaphoreType.DMA,
        pltpu.SemaphoreType.DMA,
        pltpu.SemaphoreType.DMA,
        pltpu.SemaphoreType.DMA,
    ],
    compiler_params=pltpu.CompilerParams(use_tc_tiling_on_sc=False),
)(_sc_agg_body)


_TC_ROWS = 2000


def _tc_prologue_body(x_ref, w_ref, nr_ref, xh_ref):
    xb = x_ref[...]
    xh_ref[0] = xb[:, :HFEAT_]
    xh_ref[1] = xb[:, HFEAT_:]
    nr = jnp.dot(xb, w_ref[...], preferred_element_type=jnp.float32,
                 precision=lax.Precision.HIGHEST)
    nr_ref[...] = jnp.maximum(nr, 0.0)


def _tc_prologue(x2d, W):
    return pl.pallas_call(
        _tc_prologue_body,
        grid=(N_NODES_ // _TC_ROWS,),
        in_specs=[
            pl.BlockSpec((_TC_ROWS, FEAT_), lambda i: (i, 0)),
            pl.BlockSpec((FEAT_, FEAT_), lambda i: (0, 0)),
        ],
        out_specs=[
            pl.BlockSpec((_TC_ROWS, FEAT_), lambda i: (i, 0)),
            pl.BlockSpec((NC_, _TC_ROWS, HFEAT_), lambda i: (0, i, 0)),
        ],
        out_shape=[
            jax.ShapeDtypeStruct((N_NODES_, FEAT_), jnp.float32),
            jax.ShapeDtypeStruct((NC_, N_NODES_, HFEAT_), jnp.float32),
        ],
    )(x2d, W)


def _tc_epilogue_body(nr_ref, w_ref, ps_ref, pc_ref, o_ref):
    ssum = jnp.concatenate([ps_ref[0], ps_ref[1]], axis=-1)
    cnt = pc_ref[0, :, 0:1] + pc_ref[1, :, 0:1]
    agg = ssum / jnp.maximum(cnt, 1.0)
    am = jnp.dot(agg, w_ref[...], preferred_element_type=jnp.float32,
                 precision=lax.Precision.HIGHEST)
    o_ref[...] = jnp.concatenate([nr_ref[...], jnp.maximum(am, 0.0)], axis=-1)


def _tc_epilogue(nr, W, psum, pcnt):
    return pl.pallas_call(
        _tc_epilogue_body,
        grid=(N_NODES_ // _TC_ROWS,),
        in_specs=[
            pl.BlockSpec((_TC_ROWS, FEAT_), lambda i: (i, 0)),
            pl.BlockSpec((FEAT_, FEAT_), lambda i: (0, 0)),
            pl.BlockSpec((NC_, _TC_ROWS, HFEAT_), lambda i: (0, i, 0)),
            pl.BlockSpec((NC_, _TC_ROWS, 16), lambda i: (0, i, 0)),
        ],
        out_specs=pl.BlockSpec((_TC_ROWS, 2 * FEAT_), lambda i: (i, 0)),
        out_shape=jax.ShapeDtypeStruct((N_NODES_, 2 * FEAT_), jnp.float32),
    )(nr, W, psum, pcnt)


def kernel(x, edge_dst, edge_src, W):
    x2d = x.astype(jnp.float32).reshape(N_NODES_, FEAT_)
    src = edge_src.astype(jnp.int32).reshape(NS_, CHUNKS_, CHUNK_)
    dst = edge_dst.astype(jnp.int32).reshape(NS_, CHUNKS_, CHUNK_)
    zero = jnp.zeros((ROWS_PER_TILE_, HFEAT_), jnp.float32)
    zcnt = jnp.zeros((ROWS_PER_TILE_, 16), jnp.float32)
    ones = jnp.ones((CHUNK_, 16), jnp.float32)
    nr, xh = _tc_prologue(x2d, W)
    psum, pcnt = _sc_agg(xh, src, dst, zero, zcnt, ones)
    psum = psum.reshape(NC_, N_NODES_, HFEAT_)
    pcnt = pcnt.reshape(NC_, N_NODES_, 16)
    out = _tc_epilogue(nr, W, psum, pcnt)
    return out.reshape(N_NODES_, 1, 1, 2 * FEAT_)
